# Initial kernel scaffold; baseline (speedup 1.0000x reference)
import jax
import jax.numpy as jnp
from jax.experimental import pallas as pl


def kernel(feat, coordinate, w1, b1, w2, b2, w3, b3, num_graphs):
    raise NotImplementedError("write your pallas kernel here")


# speedup vs baseline: 1.6740x; 1.6740x over previous
import jax
import jax.numpy as jnp
from jax.experimental import pallas as pl

_B = 128
_n = 32
_F = 128
_H = 64
_O = 128
_G = 8


def _body(feat_ref, x_ref, w1a_ref, w1b_ref, w1d_ref, b1_ref,
          w2_ref, b2_ref, w3_ref, b3_ref, out_ref):
    h = feat_ref[...]
    x = x_ref[...]
    a = jnp.dot(h, w1a_ref[...], preferred_element_type=jnp.float32)
    c = jnp.dot(h, w1b_ref[...], preferred_element_type=jnp.float32)
    a3 = a.reshape(_G, _n, _H)
    c3 = c.reshape(_G, _n, _H)
    x3 = x.reshape(_G, _n, 3)
    diff = x3[:, None, :, :] - x3[:, :, None, :]
    d2 = jnp.sum(diff * diff, axis=-1, keepdims=True)
    z1 = (d2 * w1d_ref[...][0]
          + a3[:, None, :, :]
          + c3[:, :, None, :]
          + b1_ref[...][0])
    h1 = jax.nn.silu(z1).reshape(_G * _n * _n, _H)
    z2 = jnp.dot(h1, w2_ref[...], preferred_element_type=jnp.float32)
    h2 = jax.nn.silu(z2 + b2_ref[...][0])
    s = jnp.sum(h2.reshape(_G, _n * _n, _H), axis=1)
    out = (jnp.dot(s, w3_ref[...], preferred_element_type=jnp.float32)
           + float(_n * _n) * b3_ref[...][0])
    out_ref[...] = out


def kernel(feat, coordinate, w1, b1, w2, b2, w3, b3, num_graphs):
    del num_graphs
    w1d = w1[0:1, :]
    w1a = w1[1:1 + _F, :]
    w1b = w1[1 + _F:, :]
    b1r = b1.reshape(1, _H)
    b2r = b2.reshape(1, _H)
    b3r = b3.reshape(1, _O)

    grid = _B // _G
    out = pl.pallas_call(
        _body,
        grid=(grid,),
        in_specs=[
            pl.BlockSpec((_G * _n, _F), lambda g: (g, 0)),
            pl.BlockSpec((_G * _n, 3), lambda g: (g, 0)),
            pl.BlockSpec((_F, _H), lambda g: (0, 0)),
            pl.BlockSpec((_F, _H), lambda g: (0, 0)),
            pl.BlockSpec((1, _H), lambda g: (0, 0)),
            pl.BlockSpec((1, _H), lambda g: (0, 0)),
            pl.BlockSpec((_H, _H), lambda g: (0, 0)),
            pl.BlockSpec((1, _H), lambda g: (0, 0)),
            pl.BlockSpec((_H, _O), lambda g: (0, 0)),
            pl.BlockSpec((1, _O), lambda g: (0, 0)),
        ],
        out_specs=pl.BlockSpec((_G, _O), lambda g: (g, 0)),
        out_shape=jax.ShapeDtypeStruct((_B, _O), jnp.float32),
    )(feat, coordinate, w1a, w1b, w1d, b1r, w2, b2r, w3, b3r)
    return out


# speedup vs baseline: 1.9602x; 1.1709x over previous
import jax
import jax.numpy as jnp
from jax import lax
from jax.experimental import pallas as pl

_B = 128
_n = 32
_F = 128
_H = 64
_O = 128
_G = 8


def _body(feat_ref, x_ref, w1a_ref, w1b_ref, w1dh_ref, w1dm_ref, b1h_ref,
          w2h_ref, b2h_ref, w3_ref, b3s_ref, sel_ref, out_ref):
    h = feat_ref[...]
    x = x_ref[...]
    a = jnp.dot(h, w1a_ref[...], preferred_element_type=jnp.float32)
    c = jnp.dot(h, w1b_ref[...], preferred_element_type=jnp.float32)
    xx = x * x
    sqn = xx[:, 0:1] + xx[:, 1:2] + xx[:, 2:3]
    ap = a + sqn * w1dh_ref[...]
    cp = c + sqn * w1dh_ref[...] + b1h_ref[...]
    x3 = x.reshape(_G, _n, 3)
    d2c = lax.dot_general(x3, x3, (((2,), (2,)), ((0,), (0,))),
                          preferred_element_type=jnp.float32)
    zh = (ap.reshape(_G, 1, _n, _H)
          + cp.reshape(_G, _n, 1, _H)
          + d2c[:, :, :, None] * w1dm_ref[...][0])
    t1 = jnp.tanh(zh)
    h1 = (zh + zh * t1).reshape(_G * _n * _n, _H)
    z2h = jnp.dot(h1, w2h_ref[...],
                  preferred_element_type=jnp.float32) + b2h_ref[...][0]
    t2 = jnp.tanh(z2h)
    h2 = z2h + z2h * t2
    s = jnp.dot(sel_ref[...], h2,
                preferred_element_type=jnp.float32)
    out_ref[...] = (jnp.dot(s, w3_ref[...], preferred_element_type=jnp.float32)
                    + b3s_ref[...][0])


def kernel(feat, coordinate, w1, b1, w2, b2, w3, b3, num_graphs):
    del num_graphs
    w1dh = 0.5 * w1[0:1, :]
    w1dm = -1.0 * w1[0:1, :]
    w1a = 0.5 * w1[1:1 + _F, :]
    w1b = 0.5 * w1[1 + _F:, :]
    b1h = 0.5 * b1.reshape(1, _H)
    w2h = 0.5 * w2
    b2h = 0.5 * b2.reshape(1, _H)
    b3s = float(_n * _n) * b3.reshape(1, _O)
    sel = jnp.repeat(jnp.eye(_G, dtype=jnp.float32), _n * _n, axis=1)

    grid = _B // _G
    out = pl.pallas_call(
        _body,
        grid=(grid,),
        in_specs=[
            pl.BlockSpec((_G * _n, _F), lambda g: (g, 0)),
            pl.BlockSpec((_G * _n, 3), lambda g: (g, 0)),
            pl.BlockSpec((_F, _H), lambda g: (0, 0)),
            pl.BlockSpec((_F, _H), lambda g: (0, 0)),
            pl.BlockSpec((1, _H), lambda g: (0, 0)),
            pl.BlockSpec((1, _H), lambda g: (0, 0)),
            pl.BlockSpec((1, _H), lambda g: (0, 0)),
            pl.BlockSpec((_H, _H), lambda g: (0, 0)),
            pl.BlockSpec((1, _H), lambda g: (0, 0)),
            pl.BlockSpec((_H, _O), lambda g: (0, 0)),
            pl.BlockSpec((1, _O), lambda g: (0, 0)),
            pl.BlockSpec((_G, _G * _n * _n), lambda g: (0, 0)),
        ],
        out_specs=pl.BlockSpec((_G, _O), lambda g: (g, 0)),
        out_shape=jax.ShapeDtypeStruct((_B, _O), jnp.float32),
    )(feat, coordinate, w1a, w1b, w1dh, w1dm, b1h, w2h, b2h, w3, b3s, sel)
    return out


# speedup vs baseline: 2.1152x; 1.0791x over previous
import jax
import jax.numpy as jnp
from jax import lax
from jax.experimental import pallas as pl

_B = 128
_n = 32
_F = 128
_H = 64
_O = 128
_G = 16


def _body(feat_ref, x_ref, w1a_ref, w1b_ref, w1dh_ref, w1dm_ref, b1h_ref,
          w2h_ref, b2h_ref, w3_ref, b3s_ref, sel_ref, out_ref):
    h = feat_ref[...]
    x = x_ref[...]
    a = jnp.dot(h, w1a_ref[...], preferred_element_type=jnp.float32)
    c = jnp.dot(h, w1b_ref[...], preferred_element_type=jnp.float32)
    xx = x * x
    sqn = xx[:, 0:1] + xx[:, 1:2] + xx[:, 2:3]
    ap = a + sqn * w1dh_ref[...]
    cp = c + sqn * w1dh_ref[...] + b1h_ref[...]
    x3 = x.reshape(_G, _n, 3)
    d2c = lax.dot_general(x3, x3, (((2,), (2,)), ((0,), (0,))),
                          preferred_element_type=jnp.float32)
    zh = (ap.reshape(_G, 1, _n, _H)
          + cp.reshape(_G, _n, 1, _H)
          + d2c[:, :, :, None] * w1dm_ref[...][0])
    t1 = jnp.tanh(zh)
    h1 = (zh + zh * t1).reshape(_G * _n * _n, _H)
    z2h = jnp.dot(h1, w2h_ref[...],
                  preferred_element_type=jnp.float32) + b2h_ref[...][0]
    t2 = jnp.tanh(z2h)
    h2 = z2h + z2h * t2
    s = jnp.dot(sel_ref[...], h2,
                preferred_element_type=jnp.float32)
    out_ref[...] = (jnp.dot(s, w3_ref[...], preferred_element_type=jnp.float32)
                    + b3s_ref[...][0])


def kernel(feat, coordinate, w1, b1, w2, b2, w3, b3, num_graphs):
    del num_graphs
    w1dh = 0.5 * w1[0:1, :]
    w1dm = -1.0 * w1[0:1, :]
    w1a = 0.5 * w1[1:1 + _F, :]
    w1b = 0.5 * w1[1 + _F:, :]
    b1h = 0.5 * b1.reshape(1, _H)
    w2h = 0.5 * w2
    b2h = 0.5 * b2.reshape(1, _H)
    b3s = float(_n * _n) * b3.reshape(1, _O)
    sel = jnp.repeat(jnp.eye(_G, dtype=jnp.float32), _n * _n, axis=1)

    grid = _B // _G
    out = pl.pallas_call(
        _body,
        grid=(grid,),
        in_specs=[
            pl.BlockSpec((_G * _n, _F), lambda g: (g, 0)),
            pl.BlockSpec((_G * _n, 3), lambda g: (g, 0)),
            pl.BlockSpec((_F, _H), lambda g: (0, 0)),
            pl.BlockSpec((_F, _H), lambda g: (0, 0)),
            pl.BlockSpec((1, _H), lambda g: (0, 0)),
            pl.BlockSpec((1, _H), lambda g: (0, 0)),
            pl.BlockSpec((1, _H), lambda g: (0, 0)),
            pl.BlockSpec((_H, _H), lambda g: (0, 0)),
            pl.BlockSpec((1, _H), lambda g: (0, 0)),
            pl.BlockSpec((_H, _O), lambda g: (0, 0)),
            pl.BlockSpec((1, _O), lambda g: (0, 0)),
            pl.BlockSpec((_G, _G * _n * _n), lambda g: (0, 0)),
        ],
        out_specs=pl.BlockSpec((_G, _O), lambda g: (g, 0)),
        out_shape=jax.ShapeDtypeStruct((_B, _O), jnp.float32),
    )(feat, coordinate, w1a, w1b, w1dh, w1dm, b1h, w2h, b2h, w3, b3s, sel)
    return out
